# Initial kernel scaffold; baseline (speedup 1.0000x reference)
import jax
import jax.numpy as jnp
from jax.experimental import pallas as pl


def kernel(x, W_enc, codebook, W_dec):
    raise NotImplementedError("write your pallas kernel here")


# speedup vs baseline: 3.8674x; 3.8674x over previous
import functools

import jax
import jax.numpy as jnp
from jax import lax
from jax.experimental import pallas as pl
from jax.experimental.pallas import tpu as pltpu
from jax.experimental.pallas import tpu_sc as plsc

HOP_ = 80
T_ = 32768
CD_ = 64
K_ = 512
ROWS_ = 256
BLK_A = 16
BLK_C = 2048
NW_ = 32
BW_ = T_ // NW_


def _stats_body(x_ref, cb_ref, wd_ref, f0p_ref, vuvp_ref, tab_ref, m_scr, e_scr):
    i = pl.program_id(0)
    xb = x_ref[...]
    m_scr[pl.ds(i * BLK_A, BLK_A), :] = jnp.sum(xb, axis=2) / 80.0
    e_scr[pl.ds(i * BLK_A, BLK_A), :] = jnp.sum(xb * xb, axis=2) / 80.0

    @pl.when(i == (ROWS_ // BLK_A) - 1)
    def _():
        m = m_scr[...]
        e = e_scr[...]
        ebits = lax.bitcast_convert_type(e, jnp.int32)

        def bs_body(_, lohi):
            lo, hi = lohi
            mid = lo + (hi - lo) // 2
            cnt = jnp.sum((ebits <= mid).astype(jnp.int32))
            pred = cnt >= (T_ // 2)
            return (jnp.where(pred, lo, mid + 1), jnp.where(pred, mid, hi))

        lo, hi = lax.fori_loop(0, 31, bs_body, (jnp.int32(0), jnp.int32(2**31 - 1)))
        v1 = lo
        c1 = jnp.sum((ebits <= v1).astype(jnp.int32))
        minabove = jnp.min(jnp.where(ebits > v1, ebits, jnp.int32(2**31 - 1)))
        b_bits = jnp.where(c1 >= (T_ // 2) + 1, v1, minabove)
        a_val = lax.bitcast_convert_type(v1, jnp.float32)
        b_val = lax.bitcast_convert_type(b_bits, jnp.float32)
        med = a_val * 0.5 + b_val * 0.5
        thr = med * 0.5

        vuv = (e > thr).astype(jnp.float32)
        raw = 100.0 + 200.0 * jax.nn.sigmoid(m * 4.0)
        f0 = raw * vuv
        nv = jnp.maximum(jnp.sum(vuv), 1.0)
        mean = jnp.sum(jnp.where(vuv != 0, f0, 0.0)) / nv
        f0n = jnp.where(f0 != 0, f0 - mean, f0)

        f0p_ref[...] = jnp.zeros((1, T_ + 128), jnp.float32)
        vuvp_ref[...] = jnp.zeros((1, T_ + 128), jnp.float32)
        f0p_ref[:, 1:T_ + 1] = f0n.reshape(1, T_)
        vuvp_ref[:, 1:T_ + 1] = vuv.reshape(1, T_)

        dec_tab = jnp.dot(cb_ref[...], wd_ref[...],
                          preferred_element_type=jnp.float32)
        tab_ref[...] = jnp.zeros((K_ + 8, 8), jnp.float32)
        tab_ref[0:K_, 0:4] = dec_tab


def _vq_body(f0p_ref, vuvp_ref, w8_ref, cb_ref, out_ref):
    i = pl.program_id(0)
    base = pl.multiple_of(i * BLK_C, 128)
    f0w = f0p_ref[:, pl.ds(base, BLK_C + 128)]
    vuw = vuvp_ref[:, pl.ds(base, BLK_C + 128)]
    rows = [f0w[:, j:BLK_C + j] for j in range(4)]
    rows += [vuw[:, j:BLK_C + j] for j in range(4)]
    x8 = jnp.concatenate(rows, axis=0)
    ze_t = jnp.dot(w8_ref[...], x8,
                   preferred_element_type=jnp.float32)
    s_t = jnp.dot(cb_ref[...], ze_t,
                  preferred_element_type=jnp.float32)
    ze2 = jnp.sum(ze_t * ze_t, axis=0)
    c2 = jnp.sum(cb_ref[...] * cb_ref[...], axis=1)
    d_t = (ze2[None, :] - 2.0 * s_t) + c2[:, None]
    out_ref[0, 0, :] = jnp.argmin(d_t, axis=0).astype(jnp.int32)


def _dec_body(idx_hbm, tab_hbm, out_hbm, idx_v, out_v, tab_v):
    wid = lax.axis_index("s") * 2 + lax.axis_index("c")
    base = wid * BW_
    pltpu.sync_copy(tab_hbm, tab_v)
    sent = jnp.full((16,), K_, jnp.int32)
    idx_v[pl.ds(0, 16)] = sent
    idx_v[pl.ds(BW_, 16)] = sent
    idx_v[pl.ds(BW_ + 16, 16)] = sent

    @pl.when(wid == 0)
    def _():
        pltpu.sync_copy(idx_hbm.at[pl.ds(0, BW_ + 8)], idx_v.at[pl.ds(8, BW_ + 8)])

    @pl.when(wid == NW_ - 1)
    def _():
        pltpu.sync_copy(idx_hbm.at[pl.ds(T_ - BW_ - 8, BW_ + 8)],
                        idx_v.at[pl.ds(0, BW_ + 8)])

    @pl.when((wid > 0) & (wid < NW_ - 1))
    def _():
        pltpu.sync_copy(idx_hbm.at[pl.ds(base - 8, BW_ + 16)],
                        idx_v.at[pl.ds(0, BW_ + 16)])

    iota = lax.iota(jnp.int32, 16)

    def body(g, carry):
        s0 = g * 16 + 7
        acc = jnp.zeros((16,), jnp.float32)
        for j in range(4):
            iv = plsc.load_gather(idx_v, [iota + (s0 + j)])
            jv = jnp.full((16,), j, jnp.int32)
            acc = acc + plsc.load_gather(tab_v, [iv, jv])
        out_v[pl.ds(g * 16, 16)] = acc
        return carry

    lax.fori_loop(0, BW_ // 16, body, 0)
    pltpu.sync_copy(out_v, out_hbm.at[pl.ds(base, BW_)])


def _make_stats():
    return pl.pallas_call(
        _stats_body,
        grid=(ROWS_ // BLK_A,),
        in_specs=[
            pl.BlockSpec((BLK_A, 128, HOP_), lambda i: (i, 0, 0)),
            pl.BlockSpec((K_, CD_), lambda i: (0, 0)),
            pl.BlockSpec((CD_, 4), lambda i: (0, 0)),
        ],
        out_specs=[
            pl.BlockSpec((1, T_ + 128), lambda i: (0, 0)),
            pl.BlockSpec((1, T_ + 128), lambda i: (0, 0)),
            pl.BlockSpec((K_ + 8, 8), lambda i: (0, 0)),
        ],
        out_shape=[
            jax.ShapeDtypeStruct((1, T_ + 128), jnp.float32),
            jax.ShapeDtypeStruct((1, T_ + 128), jnp.float32),
            jax.ShapeDtypeStruct((K_ + 8, 8), jnp.float32),
        ],
        scratch_shapes=[
            pltpu.VMEM((ROWS_, 128), jnp.float32),
            pltpu.VMEM((ROWS_, 128), jnp.float32),
        ],
    )


def _make_vq():
    return pl.pallas_call(
        _vq_body,
        grid=(T_ // BLK_C,),
        in_specs=[
            pl.BlockSpec((1, T_ + 128), lambda i: (0, 0)),
            pl.BlockSpec((1, T_ + 128), lambda i: (0, 0)),
            pl.BlockSpec((CD_, 8), lambda i: (0, 0)),
            pl.BlockSpec((K_, CD_), lambda i: (0, 0)),
        ],
        out_specs=pl.BlockSpec((1, 1, BLK_C), lambda i: (i, 0, 0)),
        out_shape=jax.ShapeDtypeStruct((T_ // BLK_C, 1, BLK_C), jnp.int32),
    )


def _make_dec():
    return pl.kernel(
        _dec_body,
        out_type=jax.ShapeDtypeStruct((T_,), jnp.float32),
        mesh=plsc.VectorSubcoreMesh(core_axis_name="c", subcore_axis_name="s"),
        compiler_params=pltpu.CompilerParams(needs_layout_passes=False),
        scratch_types=[
            pltpu.VMEM((BW_ + 32, ), jnp.int32),
            pltpu.VMEM((BW_,), jnp.float32),
            pltpu.VMEM((K_ + 8, 8), jnp.float32),
        ],
    )


def kernel(x, W_enc, codebook, W_dec):
    xr3 = x.reshape(ROWS_, 128, HOP_)
    w8 = W_enc.reshape(CD_, 8)
    wd0 = W_dec[0]
    f0p, vuvp, tab = _make_stats()(xr3, codebook, wd0)
    idx3 = _make_vq()(f0p, vuvp, w8, codebook)
    idx = idx3.reshape(T_)
    return _make_dec()(idx, tab)


# speedup vs baseline: 4.2413x; 1.0967x over previous
import functools

import jax
import jax.numpy as jnp
from jax import lax
from jax.experimental import pallas as pl
from jax.experimental.pallas import tpu as pltpu
from jax.experimental.pallas import tpu_sc as plsc

HOP_ = 80
T_ = 32768
CD_ = 64
K_ = 512
ROWS_ = 256
BLK_A = 64
BLK_C = 4096
NW_ = 32
BW_ = T_ // NW_


def _stats_body(x_ref, cb_ref, wd_ref, f0p_ref, vuvp_ref, tab_ref, m_scr, e_scr):
    i = pl.program_id(0)
    xb = x_ref[...]
    m_scr[pl.ds(i * BLK_A, BLK_A), :] = jnp.sum(xb, axis=2) / 80.0
    e_scr[pl.ds(i * BLK_A, BLK_A), :] = jnp.sum(xb * xb, axis=2) / 80.0

    @pl.when(i == (ROWS_ // BLK_A) - 1)
    def _():
        m = m_scr[...]
        e = e_scr[...]
        ebits = lax.bitcast_convert_type(e, jnp.int32)

        def bs_body(_, lohi):
            lo, hi = lohi
            mid = lo + (hi - lo) // 2
            cnt = jnp.sum((ebits <= mid).astype(jnp.int32))
            pred = cnt >= (T_ // 2)
            return (jnp.where(pred, lo, mid + 1), jnp.where(pred, mid, hi))

        lo, hi = lax.fori_loop(0, 31, bs_body, (jnp.int32(0), jnp.int32(2**31 - 1)))
        v1 = lo
        c1 = jnp.sum((ebits <= v1).astype(jnp.int32))
        minabove = jnp.min(jnp.where(ebits > v1, ebits, jnp.int32(2**31 - 1)))
        b_bits = jnp.where(c1 >= (T_ // 2) + 1, v1, minabove)
        a_val = lax.bitcast_convert_type(v1, jnp.float32)
        b_val = lax.bitcast_convert_type(b_bits, jnp.float32)
        med = a_val * 0.5 + b_val * 0.5
        thr = med * 0.5

        vuv = (e > thr).astype(jnp.float32)
        raw = 100.0 + 200.0 * jax.nn.sigmoid(m * 4.0)
        f0 = raw * vuv
        nv = jnp.maximum(jnp.sum(vuv), 1.0)
        mean = jnp.sum(jnp.where(vuv != 0, f0, 0.0)) / nv
        f0n = jnp.where(f0 != 0, f0 - mean, f0)

        f0p_ref[...] = jnp.zeros((1, T_ + 128), jnp.float32)
        vuvp_ref[...] = jnp.zeros((1, T_ + 128), jnp.float32)
        f0p_ref[:, 1:T_ + 1] = f0n.reshape(1, T_)
        vuvp_ref[:, 1:T_ + 1] = vuv.reshape(1, T_)

        dec_tab = jnp.dot(cb_ref[...], wd_ref[...],
                          preferred_element_type=jnp.float32)
        tab_ref[...] = jnp.zeros((K_ + 8, 8), jnp.float32)
        tab_ref[0:K_, 0:4] = dec_tab


def _vq_body(f0p_ref, vuvp_ref, w8_ref, cb_ref, cbm2_ref, out_ref):
    i = pl.program_id(0)
    base = pl.multiple_of(i * BLK_C, 128)
    f0w = f0p_ref[:, pl.ds(base, BLK_C + 128)]
    vuw = vuvp_ref[:, pl.ds(base, BLK_C + 128)]
    rows = [f0w[:, j:BLK_C + j] for j in range(4)]
    rows += [vuw[:, j:BLK_C + j] for j in range(4)]
    x8 = jnp.concatenate(rows, axis=0)
    ze_t = jnp.dot(w8_ref[...], x8,
                   preferred_element_type=jnp.float32)
    s2_t = jnp.dot(cbm2_ref[...], ze_t,
                   preferred_element_type=jnp.float32)
    ze2 = jnp.sum(ze_t * ze_t, axis=0)
    c2 = jnp.sum(cb_ref[...] * cb_ref[...], axis=1)
    d_t = (ze2[None, :] + s2_t) + c2[:, None]
    out_ref[0, 0, :] = jnp.argmin(d_t, axis=0).astype(jnp.int32)


def _dec_body(idx_hbm, tab_hbm, out_hbm, idx_v, out_v, tab_v):
    wid = lax.axis_index("s") * 2 + lax.axis_index("c")
    base = wid * BW_
    pltpu.sync_copy(tab_hbm, tab_v)
    sent = jnp.full((16,), K_, jnp.int32)
    idx_v[pl.ds(0, 16)] = sent
    idx_v[pl.ds(BW_, 16)] = sent
    idx_v[pl.ds(BW_ + 16, 16)] = sent

    @pl.when(wid == 0)
    def _():
        pltpu.sync_copy(idx_hbm.at[pl.ds(0, BW_ + 8)], idx_v.at[pl.ds(8, BW_ + 8)])

    @pl.when(wid == NW_ - 1)
    def _():
        pltpu.sync_copy(idx_hbm.at[pl.ds(T_ - BW_ - 8, BW_ + 8)],
                        idx_v.at[pl.ds(0, BW_ + 8)])

    @pl.when((wid > 0) & (wid < NW_ - 1))
    def _():
        pltpu.sync_copy(idx_hbm.at[pl.ds(base - 8, BW_ + 16)],
                        idx_v.at[pl.ds(0, BW_ + 16)])

    iota = lax.iota(jnp.int32, 16)

    def body(g, carry):
        s0 = g * 16 + 7
        acc = jnp.zeros((16,), jnp.float32)
        for j in range(4):
            iv = plsc.load_gather(idx_v, [iota + (s0 + j)])
            jv = jnp.full((16,), j, jnp.int32)
            acc = acc + plsc.load_gather(tab_v, [iv, jv])
        out_v[pl.ds(g * 16, 16)] = acc
        return carry

    lax.fori_loop(0, BW_ // 16, body, 0)
    pltpu.sync_copy(out_v, out_hbm.at[pl.ds(base, BW_)])


def _make_stats():
    return pl.pallas_call(
        _stats_body,
        grid=(ROWS_ // BLK_A,),
        in_specs=[
            pl.BlockSpec((BLK_A, 128, HOP_), lambda i: (i, 0, 0)),
            pl.BlockSpec((K_, CD_), lambda i: (0, 0)),
            pl.BlockSpec((CD_, 4), lambda i: (0, 0)),
        ],
        out_specs=[
            pl.BlockSpec((1, T_ + 128), lambda i: (0, 0)),
            pl.BlockSpec((1, T_ + 128), lambda i: (0, 0)),
            pl.BlockSpec((K_ + 8, 8), lambda i: (0, 0)),
        ],
        out_shape=[
            jax.ShapeDtypeStruct((1, T_ + 128), jnp.float32),
            jax.ShapeDtypeStruct((1, T_ + 128), jnp.float32),
            jax.ShapeDtypeStruct((K_ + 8, 8), jnp.float32),
        ],
        scratch_shapes=[
            pltpu.VMEM((ROWS_, 128), jnp.float32),
            pltpu.VMEM((ROWS_, 128), jnp.float32),
        ],
    )


def _make_vq():
    return pl.pallas_call(
        _vq_body,
        grid=(T_ // BLK_C,),
        in_specs=[
            pl.BlockSpec((1, T_ + 128), lambda i: (0, 0)),
            pl.BlockSpec((1, T_ + 128), lambda i: (0, 0)),
            pl.BlockSpec((CD_, 8), lambda i: (0, 0)),
            pl.BlockSpec((K_, CD_), lambda i: (0, 0)),
            pl.BlockSpec((K_, CD_), lambda i: (0, 0)),
        ],
        out_specs=pl.BlockSpec((1, 1, BLK_C), lambda i: (i, 0, 0)),
        out_shape=jax.ShapeDtypeStruct((T_ // BLK_C, 1, BLK_C), jnp.int32),
    )


def _make_dec():
    return pl.kernel(
        _dec_body,
        out_type=jax.ShapeDtypeStruct((T_,), jnp.float32),
        mesh=plsc.VectorSubcoreMesh(core_axis_name="c", subcore_axis_name="s"),
        compiler_params=pltpu.CompilerParams(needs_layout_passes=False),
        scratch_types=[
            pltpu.VMEM((BW_ + 32, ), jnp.int32),
            pltpu.VMEM((BW_,), jnp.float32),
            pltpu.VMEM((K_ + 8, 8), jnp.float32),
        ],
    )


def kernel(x, W_enc, codebook, W_dec):
    xr3 = x.reshape(ROWS_, 128, HOP_)
    w8 = W_enc.reshape(CD_, 8)
    wd0 = W_dec[0]
    f0p, vuvp, tab = _make_stats()(xr3, codebook, wd0)
    idx3 = _make_vq()(f0p, vuvp, w8, codebook, codebook * -2.0)
    idx = idx3.reshape(T_)
    return _make_dec()(idx, tab)


# speedup vs baseline: 4.8504x; 1.1436x over previous
import functools

import jax
import jax.numpy as jnp
from jax import lax
from jax.experimental import pallas as pl
from jax.experimental.pallas import tpu as pltpu
from jax.experimental.pallas import tpu_sc as plsc

HOP_ = 80
T_ = 32768
CD_ = 64
K_ = 512
ROWS_ = 256
BLK_A = 64
SPF_ = HOP_ * 128
BLK_C = 4096
NW_ = 32
BW_ = T_ // NW_


def _stats_body(x_ref, cb_ref, wd_ref, f0p_ref, vuvp_ref, tab_ref,
                m_scr, e_scr, p_scr):
    i = pl.program_id(0)

    @pl.when(i == 0)
    def _():
        s = lax.broadcasted_iota(jnp.int32, (SPF_, 128), 0)
        f = lax.broadcasted_iota(jnp.int32, (SPF_, 128), 1)
        p_scr[...] = (s // HOP_ == f).astype(jnp.float32)

    xb = x_ref[...]
    pmat = p_scr[...]
    m_scr[pl.ds(i * BLK_A, BLK_A), :] = jnp.dot(
        xb, pmat, preferred_element_type=jnp.float32) / 80.0
    e_scr[pl.ds(i * BLK_A, BLK_A), :] = jnp.dot(
        xb * xb, pmat, preferred_element_type=jnp.float32) / 80.0

    @pl.when(i == (ROWS_ // BLK_A) - 1)
    def _():
        m = m_scr[...]
        e = e_scr[...]
        ebits = lax.bitcast_convert_type(e, jnp.int32)

        def bs_body(_, lohi):
            lo, hi = lohi
            mid = lo + (hi - lo) // 2
            cnt = jnp.sum((ebits <= mid).astype(jnp.int32))
            pred = cnt >= (T_ // 2)
            return (jnp.where(pred, lo, mid + 1), jnp.where(pred, mid, hi))

        lo, hi = lax.fori_loop(0, 31, bs_body, (jnp.int32(0), jnp.int32(2**31 - 1)))
        v1 = lo
        c1 = jnp.sum((ebits <= v1).astype(jnp.int32))
        minabove = jnp.min(jnp.where(ebits > v1, ebits, jnp.int32(2**31 - 1)))
        b_bits = jnp.where(c1 >= (T_ // 2) + 1, v1, minabove)
        a_val = lax.bitcast_convert_type(v1, jnp.float32)
        b_val = lax.bitcast_convert_type(b_bits, jnp.float32)
        med = a_val * 0.5 + b_val * 0.5
        thr = med * 0.5

        vuv = (e > thr).astype(jnp.float32)
        raw = 100.0 + 200.0 * jax.nn.sigmoid(m * 4.0)
        f0 = raw * vuv
        nv = jnp.maximum(jnp.sum(vuv), 1.0)
        mean = jnp.sum(jnp.where(vuv != 0, f0, 0.0)) / nv
        f0n = jnp.where(f0 != 0, f0 - mean, f0)

        f0p_ref[...] = jnp.zeros((1, T_ + 128), jnp.float32)
        vuvp_ref[...] = jnp.zeros((1, T_ + 128), jnp.float32)
        f0p_ref[:, 1:T_ + 1] = f0n.reshape(1, T_)
        vuvp_ref[:, 1:T_ + 1] = vuv.reshape(1, T_)

        dec_tab = jnp.dot(cb_ref[...], wd_ref[...],
                          preferred_element_type=jnp.float32)
        tab_ref[...] = jnp.zeros((K_ + 8, 8), jnp.float32)
        tab_ref[0:K_, 0:4] = dec_tab


def _vq_body(f0p_ref, vuvp_ref, w8_ref, cb_ref, cbm2_ref, out_ref):
    i = pl.program_id(0)
    base = pl.multiple_of(i * BLK_C, 128)
    f0w = f0p_ref[:, pl.ds(base, BLK_C + 128)]
    vuw = vuvp_ref[:, pl.ds(base, BLK_C + 128)]
    rows = [f0w[:, j:BLK_C + j] for j in range(4)]
    rows += [vuw[:, j:BLK_C + j] for j in range(4)]
    x8 = jnp.concatenate(rows, axis=0)
    ze_t = jnp.dot(w8_ref[...], x8,
                   preferred_element_type=jnp.float32)
    s2_t = jnp.dot(cbm2_ref[...], ze_t,
                   preferred_element_type=jnp.float32)
    ze2 = jnp.sum(ze_t * ze_t, axis=0)
    c2 = jnp.sum(cb_ref[...] * cb_ref[...], axis=1)
    d_t = (ze2[None, :] + s2_t) + c2[:, None]
    out_ref[0, 0, :] = jnp.argmin(d_t, axis=0).astype(jnp.int32)


def _dec_body(idx_hbm, tab_hbm, out_hbm, idx_v, out_v, tab_v):
    wid = lax.axis_index("s") * 2 + lax.axis_index("c")
    base = wid * BW_
    pltpu.sync_copy(tab_hbm, tab_v)
    sent = jnp.full((16,), K_, jnp.int32)
    idx_v[pl.ds(0, 16)] = sent
    idx_v[pl.ds(BW_, 16)] = sent
    idx_v[pl.ds(BW_ + 16, 16)] = sent

    @pl.when(wid == 0)
    def _():
        pltpu.sync_copy(idx_hbm.at[pl.ds(0, BW_ + 8)], idx_v.at[pl.ds(8, BW_ + 8)])

    @pl.when(wid == NW_ - 1)
    def _():
        pltpu.sync_copy(idx_hbm.at[pl.ds(T_ - BW_ - 8, BW_ + 8)],
                        idx_v.at[pl.ds(0, BW_ + 8)])

    @pl.when((wid > 0) & (wid < NW_ - 1))
    def _():
        pltpu.sync_copy(idx_hbm.at[pl.ds(base - 8, BW_ + 16)],
                        idx_v.at[pl.ds(0, BW_ + 16)])

    iota = lax.iota(jnp.int32, 16)

    def body(g, carry):
        s0 = g * 16 + 7
        acc = jnp.zeros((16,), jnp.float32)
        for j in range(4):
            iv = plsc.load_gather(idx_v, [iota + (s0 + j)])
            jv = jnp.full((16,), j, jnp.int32)
            acc = acc + plsc.load_gather(tab_v, [iv, jv])
        out_v[pl.ds(g * 16, 16)] = acc
        return carry

    lax.fori_loop(0, BW_ // 16, body, 0)
    pltpu.sync_copy(out_v, out_hbm.at[pl.ds(base, BW_)])


def _make_stats():
    return pl.pallas_call(
        _stats_body,
        grid=(ROWS_ // BLK_A,),
        in_specs=[
            pl.BlockSpec((BLK_A, SPF_), lambda i: (i, 0)),
            pl.BlockSpec((K_, CD_), lambda i: (0, 0)),
            pl.BlockSpec((CD_, 4), lambda i: (0, 0)),
        ],
        out_specs=[
            pl.BlockSpec((1, T_ + 128), lambda i: (0, 0)),
            pl.BlockSpec((1, T_ + 128), lambda i: (0, 0)),
            pl.BlockSpec((K_ + 8, 8), lambda i: (0, 0)),
        ],
        out_shape=[
            jax.ShapeDtypeStruct((1, T_ + 128), jnp.float32),
            jax.ShapeDtypeStruct((1, T_ + 128), jnp.float32),
            jax.ShapeDtypeStruct((K_ + 8, 8), jnp.float32),
        ],
        scratch_shapes=[
            pltpu.VMEM((ROWS_, 128), jnp.float32),
            pltpu.VMEM((ROWS_, 128), jnp.float32),
            pltpu.VMEM((SPF_, 128), jnp.float32),
        ],
    )


def _make_vq():
    return pl.pallas_call(
        _vq_body,
        grid=(T_ // BLK_C,),
        in_specs=[
            pl.BlockSpec((1, T_ + 128), lambda i: (0, 0)),
            pl.BlockSpec((1, T_ + 128), lambda i: (0, 0)),
            pl.BlockSpec((CD_, 8), lambda i: (0, 0)),
            pl.BlockSpec((K_, CD_), lambda i: (0, 0)),
            pl.BlockSpec((K_, CD_), lambda i: (0, 0)),
        ],
        out_specs=pl.BlockSpec((1, 1, BLK_C), lambda i: (i, 0, 0)),
        out_shape=jax.ShapeDtypeStruct((T_ // BLK_C, 1, BLK_C), jnp.int32),
    )


def _make_dec():
    return pl.kernel(
        _dec_body,
        out_type=jax.ShapeDtypeStruct((T_,), jnp.float32),
        mesh=plsc.VectorSubcoreMesh(core_axis_name="c", subcore_axis_name="s"),
        compiler_params=pltpu.CompilerParams(needs_layout_passes=False),
        scratch_types=[
            pltpu.VMEM((BW_ + 32, ), jnp.int32),
            pltpu.VMEM((BW_,), jnp.float32),
            pltpu.VMEM((K_ + 8, 8), jnp.float32),
        ],
    )


def kernel(x, W_enc, codebook, W_dec):
    x2 = x.reshape(ROWS_, SPF_)
    w8 = W_enc.reshape(CD_, 8)
    wd0 = W_dec[0]
    f0p, vuvp, tab = _make_stats()(x2, codebook, wd0)
    idx3 = _make_vq()(f0p, vuvp, w8, codebook, codebook * -2.0)
    idx = idx3.reshape(T_)
    return _make_dec()(idx, tab)
